# baseline (speedup 1.0000x reference)
import jax
import jax.numpy as jnp
from jax import lax
from jax.experimental import pallas as pl
from jax.experimental.pallas import tpu as pltpu
from jax.experimental.pallas import tpu_sc as plsc

B, F, T = 128, 80, 4096
NFREQ, NTIME = 2, 10
NW = 32
BPW = B // NW
GPB = F // 8
HT = T // 2
UPB = GPB * 2
NU = BPW * UPB
NSLOT = 6
NITER = (NU + NSLOT - 1) // NSLOT
SM_XL, SM_TS, SM_TL, SM_SA, SM_SE = 0, 8, 18, 28, 38


def _sc_body(x_hbm, prm_hbm, out_hbm, prm_v, zgrp_v,
             r0, r1, r2, r3, r4, r5, sm,
             si0, si1, si2, si3, si4, si5,
             so0, so1, so2, so3, so4, so5, sem_z):
    cid = lax.axis_index("c")
    sid = lax.axis_index("s")
    wid = sid * 2 + cid

    rbufs = (r0, r1, r2, r3, r4, r5)
    sin = (si0, si1, si2, si3, si4, si5)
    sout = (so0, so1, so2, so3, so4, so5)

    pltpu.sync_copy(prm_hbm, prm_v)

    z16 = jnp.zeros((16,), jnp.float32)
    i16 = lax.broadcasted_iota(jnp.int32, (16,), 0)

    def _zg(i, c):
        zgrp_v[i // (HT // 16), pl.ds((i % (HT // 16)) * 16, 16)] = z16
        return c
    lax.fori_loop(0, 8 * (HT // 16), _zg, 0)

    xlv = prm_v[wid]
    fv = prm_v[32]
    tsv = prm_v[33]
    tlv = prm_v[34]
    for j in range(BPW):
        sm[SM_XL + j] = xlv[j]
    for i in range(NTIME):
        sm[SM_TS + i] = tsv[i]
        sm[SM_TL + i] = tlv[i]

    s0 = fv[0]
    s1 = fv[1]
    e0 = s0 + fv[2]
    e1 = s1 + fv[3]
    p = s1 < s0
    A0 = jnp.where(p, s1, s0)
    E0 = jnp.where(p, e1, e0)
    A1 = jnp.where(p, s0, s1)
    E1 = jnp.where(p, e0, e1)
    mg = A1 <= E0
    E0 = jnp.where(mg, jnp.maximum(E0, E1), E0)
    A1 = jnp.where(mg, F, A1)
    E1 = jnp.where(mg, F, E1)

    def unit_bgh(n):
        j = n // UPB
        g = (n % UPB) // 2
        h = n % 2
        return wid * BPW + j, g * 8, h * HT

    def grp_full(n):
        g8 = ((n % UPB) // 2) * 8
        return ((g8 >= A0) & (g8 + 8 <= E0)) | ((g8 >= A1) & (g8 + 8 <= E1))

    def row_masked(f):
        return ((f >= A0) & (f < E0)) | ((f >= A1) & (f < E1))

    def issue_load(n, s):
        b, g8, c0 = unit_bgh(n)
        pltpu.make_async_copy(
            x_hbm.at[b, pl.ds(g8, 8), pl.ds(c0, HT)], rbufs[s], sin[s]
        ).start()

    def zero_span(row, sa, se):
        a0 = ((sa + 15) // 16) * 16
        a1 = (se // 16) * 16

        @pl.when((sa < a0) & (sa < se))
        def _():
            cs = a0 - 16
            v = row[pl.ds(cs, 16)]
            lane = i16 + cs
            row[pl.ds(cs, 16)] = jnp.where((lane >= sa) & (lane < se),
                                           jnp.float32(0.0), v)

        ni = jnp.maximum((a1 - a0) // 16, 0)

        def _int(k, c):
            row[pl.ds(a0 + k * 16, 16)] = z16
            return c
        lax.fori_loop(0, ni, _int, 0)

        @pl.when((a1 >= a0) & (a1 < se))
        def _():
            v = row[pl.ds(a1, 16)]
            lane = i16 + a1
            row[pl.ds(a1, 16)] = jnp.where(lane < se, jnp.float32(0.0), v)

    for s in range(NSLOT):
        @pl.when(~grp_full(jnp.int32(s)))
        def _(s=s):
            issue_load(jnp.int32(s), s)

    def iter_body(m, carry):
        o = list(carry[:NSLOT])
        nz = carry[NSLOT]
        for s in range(NSLOT):
            n = m * NSLOT + s

            @pl.when(n < NU)
            def _(n=n, s=s):
                b, g8, c0 = unit_bgh(n)
                j = n // UPB
                xl = sm[SM_XL + j]

                @pl.when(n % UPB == 0)
                def _():
                    def clip(i, c):
                        ts = sm[SM_TS + i]
                        tl = sm[SM_TL + i]
                        sm[SM_SA + i] = jnp.minimum(ts, xl)
                        sm[SM_SE + i] = jnp.minimum(ts + tl, xl)
                        return c
                    lax.fori_loop(0, NTIME, clip, 0)

                full = grp_full(n)

                @pl.when(full)
                def _():
                    pltpu.make_async_copy(
                        zgrp_v, out_hbm.at[b, pl.ds(g8, 8), pl.ds(c0, HT)],
                        sem_z).start()

                @pl.when(~full)
                def _():
                    pltpu.make_async_copy(
                        x_hbm.at[0, pl.ds(0, 8), pl.ds(0, HT)], rbufs[s],
                        sin[s]).wait()
                    for r in range(8):
                        f = g8 + r
                        row = rbufs[s].at[r]

                        @pl.when(row_masked(f))
                        def _(row=row):
                            def _zr(k, c):
                                row[pl.ds(k * 16, 16)] = z16
                                return c
                            lax.fori_loop(0, HT // 16, _zr, 0)

                        @pl.when(~row_masked(f))
                        def _(row=row, c0=c0):
                            def _sp(i, c):
                                sa = jnp.clip(sm[SM_SA + i] - c0, 0, HT)
                                se = jnp.clip(sm[SM_SE + i] - c0, 0, HT)
                                zero_span(row, sa, se)
                                return c
                            lax.fori_loop(0, NTIME, _sp, 0)
                    pltpu.make_async_copy(
                        rbufs[s], out_hbm.at[b, pl.ds(g8, 8), pl.ds(c0, HT)],
                        sout[s]).start()

            nz = nz + jnp.where((n < NU) & grp_full(n), 1, 0)
            o[s] = jnp.where((n < NU) & ~grp_full(n), 1, o[s])

        for s in range(NSLOT):
            n2 = (m + 1) * NSLOT + s

            @pl.when((n2 < NU) & ~grp_full(n2))
            def _(n2=n2, s=s):
                @pl.when(o[s] > 0)
                def _():
                    pltpu.make_async_copy(
                        x_hbm.at[0, pl.ds(0, 8), pl.ds(0, HT)], rbufs[s],
                        sout[s]).wait()
                issue_load(n2, s)
            o[s] = jnp.where((n2 < NU) & ~grp_full(n2), 0, o[s])

        return (*o, nz)

    carry = lax.fori_loop(0, NITER, iter_body, (0,) * NSLOT + (0,))

    for s in range(NSLOT):
        @pl.when(carry[s] > 0)
        def _(s=s):
            pltpu.make_async_copy(
                x_hbm.at[0, pl.ds(0, 8), pl.ds(0, HT)], rbufs[s], sout[s]
            ).wait()

    def drz(i, c):
        pltpu.make_async_copy(
            x_hbm.at[0, pl.ds(0, 8), pl.ds(0, HT)], zgrp_v, sem_z
        ).wait()
        return c
    lax.fori_loop(0, carry[NSLOT], drz, 0)


def kernel(x, x_len, freq_starts, freq_lengths, time_starts, time_lengths):
    pm = jnp.zeros((35, 16), jnp.int32)
    pm = pm.at[:NW, :BPW].set(x_len.astype(jnp.int32).reshape(NW, BPW))
    pm = pm.at[NW, :NFREQ].set(freq_starts.astype(jnp.int32))
    pm = pm.at[NW, NFREQ:2 * NFREQ].set(freq_lengths.astype(jnp.int32))
    pm = pm.at[NW + 1, :NTIME].set(time_starts.astype(jnp.int32))
    pm = pm.at[NW + 2, :NTIME].set(time_lengths.astype(jnp.int32))
    mesh = plsc.VectorSubcoreMesh(core_axis_name="c", subcore_axis_name="s")
    f = pl.kernel(
        _sc_body,
        out_type=jax.ShapeDtypeStruct((B, F, T), jnp.float32),
        mesh=mesh,
        scratch_types=[
            pltpu.VMEM((35, 16), jnp.int32),
            pltpu.VMEM((8, HT), jnp.float32),
            pltpu.VMEM((8, HT), jnp.float32),
            pltpu.VMEM((8, HT), jnp.float32),
            pltpu.VMEM((8, HT), jnp.float32),
            pltpu.VMEM((8, HT), jnp.float32),
            pltpu.VMEM((8, HT), jnp.float32),
            pltpu.VMEM((8, HT), jnp.float32),
            pltpu.SMEM((64,), jnp.int32),
            pltpu.SemaphoreType.DMA,
            pltpu.SemaphoreType.DMA,
            pltpu.SemaphoreType.DMA,
            pltpu.SemaphoreType.DMA,
            pltpu.SemaphoreType.DMA,
            pltpu.SemaphoreType.DMA,
            pltpu.SemaphoreType.DMA,
            pltpu.SemaphoreType.DMA,
            pltpu.SemaphoreType.DMA,
            pltpu.SemaphoreType.DMA,
            pltpu.SemaphoreType.DMA,
            pltpu.SemaphoreType.DMA,
            pltpu.SemaphoreType.DMA,
        ],
    )
    return f(x, pm)


# speedup vs baseline: 1.4071x; 1.4071x over previous
import jax
import jax.numpy as jnp
from jax import lax
from jax.experimental import pallas as pl
from jax.experimental.pallas import tpu as pltpu
from jax.experimental.pallas import tpu_sc as plsc

B, F, T = 128, 80, 4096
NFREQ, NTIME = 2, 10
NW = 32
BPW = B // NW
GPB = F // 8
HT = T // 2
UPB = GPB * 2
NU = BPW * UPB
NSLOT = 6
NITER = (NU + NSLOT - 1) // NSLOT
SM_XL, SM_TS, SM_TL, SM_SA, SM_SE = 0, 8, 18, 28, 38


def _sc_body(x_hbm, prm_hbm, out_hbm, prm_v, zgrp_v,
             r0, r1, r2, r3, r4, r5, sm,
             si0, si1, si2, si3, si4, si5,
             so0, so1, so2, so3, so4, so5, sem_z):
    cid = lax.axis_index("c")
    sid = lax.axis_index("s")
    wid = sid * 2 + cid

    rbufs = (r0, r1, r2, r3, r4, r5)
    sin = (si0, si1, si2, si3, si4, si5)
    sout = (so0, so1, so2, so3, so4, so5)

    pltpu.sync_copy(prm_hbm, prm_v)

    z16 = jnp.zeros((16,), jnp.float32)
    i16 = lax.broadcasted_iota(jnp.int32, (16,), 0)

    def _zg(i, c):
        zgrp_v[i // (HT // 16), pl.ds((i % (HT // 16)) * 16, 16)] = z16
        return c
    lax.fori_loop(0, 8 * (HT // 16), _zg, 0)

    xlv = prm_v[wid]
    fv = prm_v[32]
    tsv = prm_v[33]
    tlv = prm_v[34]
    for j in range(BPW):
        sm[SM_XL + j] = xlv[j]
    for i in range(NTIME):
        sm[SM_TS + i] = tsv[i]
        sm[SM_TL + i] = tlv[i]

    s0 = fv[0]
    s1 = fv[1]
    e0 = s0 + fv[2]
    e1 = s1 + fv[3]
    p = s1 < s0
    A0 = jnp.where(p, s1, s0)
    E0 = jnp.where(p, e1, e0)
    A1 = jnp.where(p, s0, s1)
    E1 = jnp.where(p, e0, e1)
    mg = A1 <= E0
    E0 = jnp.where(mg, jnp.maximum(E0, E1), E0)
    A1 = jnp.where(mg, F, A1)
    E1 = jnp.where(mg, F, E1)

    def unit_bgh(n):
        j = n // UPB
        g = (n % UPB) // 2
        h = n % 2
        return wid * BPW + j, g * 8, h * HT

    def grp_full(n):
        g8 = ((n % UPB) // 2) * 8
        return ((g8 >= A0) & (g8 + 8 <= E0)) | ((g8 >= A1) & (g8 + 8 <= E1))

    def row_masked(f):
        return ((f >= A0) & (f < E0)) | ((f >= A1) & (f < E1))

    def issue_load(n, s):
        b, g8, c0 = unit_bgh(n)
        pltpu.make_async_copy(
            x_hbm.at[b, pl.ds(g8, 8), pl.ds(c0, HT)], rbufs[s], sin[s]
        ).start()

    def zero_span_vst(row, sa, se, a0, kpos, rem):
        @pl.when(sa < a0)
        def _():
            cs = a0 - 16
            v = row[pl.ds(cs, 16)]
            lane = i16 + cs
            row[pl.ds(cs, 16)] = jnp.where((lane >= sa) & (lane < se),
                                           jnp.float32(0.0), v)

        def _ik(q, c2):
            row[pl.ds(a0 + q * 16, 16)] = z16
            return c2
        lax.fori_loop(0, kpos, _ik, 0)

        t0 = a0 + 16 * kpos

        @pl.when(t0 < se)
        def _():
            cs = jnp.minimum(t0, HT - 16)
            v = row[pl.ds(cs, 16)]
            lane = i16 + cs
            row[pl.ds(cs, 16)] = jnp.where((lane >= sa) & (lane < se),
                                           jnp.float32(0.0), v)

    for s in range(NSLOT):
        @pl.when(~grp_full(jnp.int32(s)))
        def _(s=s):
            issue_load(jnp.int32(s), s)

    def iter_body(m, carry):
        o = list(carry[:NSLOT])
        nz = carry[NSLOT]
        for s in range(NSLOT):
            n = m * NSLOT + s

            @pl.when(n < NU)
            def _(n=n, s=s):
                b, g8, c0 = unit_bgh(n)
                j = n // UPB
                xl = sm[SM_XL + j]

                @pl.when(n % UPB == 0)
                def _():
                    def clip(i, c):
                        ts = sm[SM_TS + i]
                        tl = sm[SM_TL + i]
                        sm[SM_SA + i] = jnp.minimum(ts, xl)
                        sm[SM_SE + i] = jnp.minimum(ts + tl, xl)
                        return c
                    lax.fori_loop(0, NTIME, clip, 0)

                full = grp_full(n)

                @pl.when(full)
                def _():
                    pltpu.make_async_copy(
                        zgrp_v, out_hbm.at[b, pl.ds(g8, 8), pl.ds(c0, HT)],
                        sem_z).start()

                @pl.when(~full)
                def _():
                    pltpu.make_async_copy(
                        x_hbm.at[0, pl.ds(0, 8), pl.ds(0, HT)], rbufs[s],
                        sin[s]).wait()

                    for r in range(8):
                        @pl.when(row_masked(g8 + r))
                        def _(r=r):
                            row = rbufs[s].at[r]

                            def _zr(kk, c):
                                base = kk * 128
                                for u in range(8):
                                    row[pl.ds(base + u * 16, 16)] = z16
                                return c
                            lax.fori_loop(0, HT // 128, _zr, 0)

                    def _sp(i, c):
                        sa = jnp.clip(sm[SM_SA + i] - c0, 0, HT)
                        se = jnp.clip(sm[SM_SE + i] - c0, 0, HT)
                        rem = se - sa

                        @pl.when(rem > 0)
                        def _():
                            a0 = ((sa + 15) // 16) * 16
                            kpos = jnp.maximum((se - a0) // 16, 0)
                            for r in range(8):
                                @pl.when(~row_masked(g8 + r))
                                def _(r=r):
                                    zero_span_vst(rbufs[s].at[r], sa, se,
                                                  a0, kpos, rem)
                        return c
                    lax.fori_loop(0, NTIME, _sp, 0)

                    pltpu.make_async_copy(
                        rbufs[s], out_hbm.at[b, pl.ds(g8, 8), pl.ds(c0, HT)],
                        sout[s]).start()

            nz = nz + jnp.where((n < NU) & grp_full(n), 1, 0)
            o[s] = jnp.where((n < NU) & ~grp_full(n), 1, o[s])

        for s in range(NSLOT):
            n2 = (m + 1) * NSLOT + s

            @pl.when((n2 < NU) & ~grp_full(n2))
            def _(n2=n2, s=s):
                @pl.when(o[s] > 0)
                def _():
                    pltpu.make_async_copy(
                        x_hbm.at[0, pl.ds(0, 8), pl.ds(0, HT)], rbufs[s],
                        sout[s]).wait()
                issue_load(n2, s)
            o[s] = jnp.where((n2 < NU) & ~grp_full(n2), 0, o[s])

        return (*o, nz)

    carry = lax.fori_loop(0, NITER, iter_body, (0,) * NSLOT + (0,))

    for s in range(NSLOT):
        @pl.when(carry[s] > 0)
        def _(s=s):
            pltpu.make_async_copy(
                x_hbm.at[0, pl.ds(0, 8), pl.ds(0, HT)], rbufs[s], sout[s]
            ).wait()

    def drz(i, c):
        pltpu.make_async_copy(
            x_hbm.at[0, pl.ds(0, 8), pl.ds(0, HT)], zgrp_v, sem_z
        ).wait()
        return c
    lax.fori_loop(0, carry[NSLOT], drz, 0)


def kernel(x, x_len, freq_starts, freq_lengths, time_starts, time_lengths):
    pm = jnp.zeros((35, 16), jnp.int32)
    pm = pm.at[:NW, :BPW].set(x_len.astype(jnp.int32).reshape(NW, BPW))
    pm = pm.at[NW, :NFREQ].set(freq_starts.astype(jnp.int32))
    pm = pm.at[NW, NFREQ:2 * NFREQ].set(freq_lengths.astype(jnp.int32))
    pm = pm.at[NW + 1, :NTIME].set(time_starts.astype(jnp.int32))
    pm = pm.at[NW + 2, :NTIME].set(time_lengths.astype(jnp.int32))
    mesh = plsc.VectorSubcoreMesh(core_axis_name="c", subcore_axis_name="s")
    f = pl.kernel(
        _sc_body,
        out_type=jax.ShapeDtypeStruct((B, F, T), jnp.float32),
        mesh=mesh,
        scratch_types=[
            pltpu.VMEM((35, 16), jnp.int32),
            pltpu.VMEM((8, HT), jnp.float32),
            pltpu.VMEM((8, HT), jnp.float32),
            pltpu.VMEM((8, HT), jnp.float32),
            pltpu.VMEM((8, HT), jnp.float32),
            pltpu.VMEM((8, HT), jnp.float32),
            pltpu.VMEM((8, HT), jnp.float32),
            pltpu.VMEM((8, HT), jnp.float32),
            pltpu.SMEM((64,), jnp.int32),
            pltpu.SemaphoreType.DMA,
            pltpu.SemaphoreType.DMA,
            pltpu.SemaphoreType.DMA,
            pltpu.SemaphoreType.DMA,
            pltpu.SemaphoreType.DMA,
            pltpu.SemaphoreType.DMA,
            pltpu.SemaphoreType.DMA,
            pltpu.SemaphoreType.DMA,
            pltpu.SemaphoreType.DMA,
            pltpu.SemaphoreType.DMA,
            pltpu.SemaphoreType.DMA,
            pltpu.SemaphoreType.DMA,
            pltpu.SemaphoreType.DMA,
        ],
    )
    return f(x, pm)


# speedup vs baseline: 1.5636x; 1.1113x over previous
import jax
import jax.numpy as jnp
from jax import lax
from jax.experimental import pallas as pl
from jax.experimental.pallas import tpu as pltpu
from jax.experimental.pallas import tpu_sc as plsc

B, F, T = 128, 80, 4096
NFREQ, NTIME = 2, 10
NW = 32
BPW = B // NW
GPB = F // 8
HT = T // 2
UPB = GPB
NU = BPW * UPB
NSLOT = 3
NITER = (NU + NSLOT - 1) // NSLOT
SM_XL, SM_TS, SM_TL, SM_SA, SM_SE = 0, 8, 18, 28, 38


def _sc_body(x_hbm, prm_hbm, out_hbm, prm_v, zgrp_v,
             r0, r1, r2, sm,
             si0, si1, si2,
             so0, so1, so2, sem_z):
    cid = lax.axis_index("c")
    sid = lax.axis_index("s")
    wid = sid * 2 + cid

    rbufs = (r0, r1, r2)
    sin = (si0, si1, si2)
    sout = (so0, so1, so2)

    pltpu.sync_copy(prm_hbm, prm_v)

    z16 = jnp.zeros((16,), jnp.float32)
    i16 = lax.broadcasted_iota(jnp.int32, (16,), 0)

    def _zg(i, c):
        zgrp_v[i // (HT // 16), pl.ds((i % (HT // 16)) * 16, 16)] = z16
        return c
    lax.fori_loop(0, 8 * (HT // 16), _zg, 0)

    xlv = prm_v[wid]
    fv = prm_v[32]
    tsv = prm_v[33]
    tlv = prm_v[34]
    for j in range(BPW):
        sm[SM_XL + j] = xlv[j]
    for i in range(NTIME):
        sm[SM_TS + i] = tsv[i]
        sm[SM_TL + i] = tlv[i]

    s0 = fv[0]
    s1 = fv[1]
    e0 = s0 + fv[2]
    e1 = s1 + fv[3]
    p = s1 < s0
    A0 = jnp.where(p, s1, s0)
    E0 = jnp.where(p, e1, e0)
    A1 = jnp.where(p, s0, s1)
    E1 = jnp.where(p, e0, e1)
    mg = A1 <= E0
    E0 = jnp.where(mg, jnp.maximum(E0, E1), E0)
    A1 = jnp.where(mg, F, A1)
    E1 = jnp.where(mg, F, E1)

    def unit_bgh(n):
        j = n // UPB
        g = n % UPB
        return wid * BPW + j, g * 8

    def grp_full(n):
        g8 = (n % UPB) * 8
        return ((g8 >= A0) & (g8 + 8 <= E0)) | ((g8 >= A1) & (g8 + 8 <= E1))

    def row_masked(f):
        return ((f >= A0) & (f < E0)) | ((f >= A1) & (f < E1))

    def issue_load(n, s):
        b, g8 = unit_bgh(n)
        pltpu.make_async_copy(
            x_hbm.at[b, pl.ds(g8, 8), :], rbufs[s], sin[s]
        ).start()

    def zero_span_vst(row, sa, se, a0, kpos, rem):
        @pl.when(sa < a0)
        def _():
            cs = a0 - 16
            v = row[pl.ds(cs, 16)]
            lane = i16 + cs
            row[pl.ds(cs, 16)] = jnp.where((lane >= sa) & (lane < se),
                                           jnp.float32(0.0), v)

        def _ik(q, c2):
            row[pl.ds(a0 + q * 16, 16)] = z16
            return c2
        lax.fori_loop(0, kpos, _ik, 0)

        t0 = a0 + 16 * kpos

        @pl.when(t0 < se)
        def _():
            cs = jnp.minimum(t0, T - 16)
            v = row[pl.ds(cs, 16)]
            lane = i16 + cs
            row[pl.ds(cs, 16)] = jnp.where((lane >= sa) & (lane < se),
                                           jnp.float32(0.0), v)

    for s in range(NSLOT):
        @pl.when(~grp_full(jnp.int32(s)))
        def _(s=s):
            issue_load(jnp.int32(s), s)

    def iter_body(m, carry):
        o = list(carry[:NSLOT])
        nz = carry[NSLOT]
        for s in range(NSLOT):
            n = m * NSLOT + s

            @pl.when(n < NU)
            def _(n=n, s=s):
                b, g8 = unit_bgh(n)
                j = n // UPB
                xl = sm[SM_XL + j]

                @pl.when(n % UPB == 0)
                def _():
                    def clip(i, c):
                        ts = sm[SM_TS + i]
                        tl = sm[SM_TL + i]
                        sm[SM_SA + i] = jnp.minimum(ts, xl)
                        sm[SM_SE + i] = jnp.minimum(ts + tl, xl)
                        return c
                    lax.fori_loop(0, NTIME, clip, 0)

                full = grp_full(n)

                @pl.when(full)
                def _():
                    pltpu.make_async_copy(
                        zgrp_v, out_hbm.at[b, pl.ds(g8, 8), pl.ds(0, HT)],
                        sem_z).start()
                    pltpu.make_async_copy(
                        zgrp_v, out_hbm.at[b, pl.ds(g8, 8), pl.ds(HT, HT)],
                        sem_z).start()

                @pl.when(~full)
                def _():
                    pltpu.make_async_copy(
                        x_hbm.at[0, pl.ds(0, 8), :], rbufs[s],
                        sin[s]).wait()

                    for r in range(8):
                        @pl.when(row_masked(g8 + r))
                        def _(r=r):
                            row = rbufs[s].at[r]

                            def _zr(kk, c):
                                base = kk * 128
                                for u in range(8):
                                    row[pl.ds(base + u * 16, 16)] = z16
                                return c
                            lax.fori_loop(0, T // 128, _zr, 0)

                    def _sp(i, c):
                        sa = sm[SM_SA + i]
                        se = sm[SM_SE + i]
                        rem = se - sa

                        @pl.when(rem > 0)
                        def _():
                            a0 = ((sa + 15) // 16) * 16
                            kpos = jnp.maximum((se - a0) // 16, 0)
                            for r in range(8):
                                @pl.when(~row_masked(g8 + r))
                                def _(r=r):
                                    zero_span_vst(rbufs[s].at[r], sa, se,
                                                  a0, kpos, rem)
                        return c
                    lax.fori_loop(0, NTIME, _sp, 0)

                    pltpu.make_async_copy(
                        rbufs[s], out_hbm.at[b, pl.ds(g8, 8), :],
                        sout[s]).start()

            nz = nz + jnp.where((n < NU) & grp_full(n), 2, 0)
            o[s] = jnp.where((n < NU) & ~grp_full(n), 1, o[s])

        for s in range(NSLOT):
            n2 = (m + 1) * NSLOT + s

            @pl.when((n2 < NU) & ~grp_full(n2))
            def _(n2=n2, s=s):
                @pl.when(o[s] > 0)
                def _():
                    pltpu.make_async_copy(
                        x_hbm.at[0, pl.ds(0, 8), :], rbufs[s],
                        sout[s]).wait()
                issue_load(n2, s)
            o[s] = jnp.where((n2 < NU) & ~grp_full(n2), 0, o[s])

        return (*o, nz)

    carry = lax.fori_loop(0, NITER, iter_body, (0,) * NSLOT + (0,))

    for s in range(NSLOT):
        @pl.when(carry[s] > 0)
        def _(s=s):
            pltpu.make_async_copy(
                x_hbm.at[0, pl.ds(0, 8), :], rbufs[s], sout[s]
            ).wait()

    def drz(i, c):
        pltpu.make_async_copy(
            x_hbm.at[0, pl.ds(0, 8), pl.ds(0, HT)], zgrp_v, sem_z
        ).wait()
        return c
    lax.fori_loop(0, carry[NSLOT], drz, 0)


def kernel(x, x_len, freq_starts, freq_lengths, time_starts, time_lengths):
    pm = jnp.zeros((35, 16), jnp.int32)
    pm = pm.at[:NW, :BPW].set(x_len.astype(jnp.int32).reshape(NW, BPW))
    pm = pm.at[NW, :NFREQ].set(freq_starts.astype(jnp.int32))
    pm = pm.at[NW, NFREQ:2 * NFREQ].set(freq_lengths.astype(jnp.int32))
    pm = pm.at[NW + 1, :NTIME].set(time_starts.astype(jnp.int32))
    pm = pm.at[NW + 2, :NTIME].set(time_lengths.astype(jnp.int32))
    mesh = plsc.VectorSubcoreMesh(core_axis_name="c", subcore_axis_name="s")
    f = pl.kernel(
        _sc_body,
        out_type=jax.ShapeDtypeStruct((B, F, T), jnp.float32),
        mesh=mesh,
        scratch_types=[
            pltpu.VMEM((35, 16), jnp.int32),
            pltpu.VMEM((8, HT), jnp.float32),
            pltpu.VMEM((8, T), jnp.float32),
            pltpu.VMEM((8, T), jnp.float32),
            pltpu.VMEM((8, T), jnp.float32),
            pltpu.SMEM((64,), jnp.int32),
            pltpu.SemaphoreType.DMA,
            pltpu.SemaphoreType.DMA,
            pltpu.SemaphoreType.DMA,
            pltpu.SemaphoreType.DMA,
            pltpu.SemaphoreType.DMA,
            pltpu.SemaphoreType.DMA,
            pltpu.SemaphoreType.DMA,
        ],
    )
    return f(x, pm)


# speedup vs baseline: 1.5794x; 1.0101x over previous
import jax
import jax.numpy as jnp
from jax import lax
from jax.experimental import pallas as pl
from jax.experimental.pallas import tpu as pltpu
from jax.experimental.pallas import tpu_sc as plsc

B, F, T = 128, 80, 4096
NFREQ, NTIME = 2, 10
NW = 32
BPW = B // NW
GPB = F // 8
HT = T // 2
UPB = GPB
NU = BPW * UPB
NSLOT = 3
NITER = (NU + NSLOT - 1) // NSLOT
SM_XL, SM_TS, SM_TL, SM_SA, SM_SE = 0, 8, 18, 28, 38


def _sc_body(x_hbm, prm_hbm, out_hbm, prm_v, zgrp_v,
             r0, r1, r2, sm,
             si0, si1, si2,
             so0, so1, so2, sem_z):
    cid = lax.axis_index("c")
    sid = lax.axis_index("s")
    wid = sid * 2 + cid

    rbufs = (r0, r1, r2)
    sin = (si0, si1, si2)
    sout = (so0, so1, so2)

    pltpu.sync_copy(prm_hbm, prm_v)

    z16 = jnp.zeros((16,), jnp.float32)
    i16 = lax.broadcasted_iota(jnp.int32, (16,), 0)

    def _zg(i, c):
        zgrp_v[i // (HT // 16), pl.ds((i % (HT // 16)) * 16, 16)] = z16
        return c
    lax.fori_loop(0, 8 * (HT // 16), _zg, 0)

    xlv = prm_v[wid]
    fv = prm_v[32]
    tsv = prm_v[33]
    tlv = prm_v[34]
    for j in range(BPW):
        sm[SM_XL + j] = xlv[j]
    for i in range(NTIME):
        sm[SM_TS + i] = tsv[i]
        sm[SM_TL + i] = tlv[i]

    s0 = fv[0]
    s1 = fv[1]
    e0 = s0 + fv[2]
    e1 = s1 + fv[3]
    p = s1 < s0
    A0 = jnp.where(p, s1, s0)
    E0 = jnp.where(p, e1, e0)
    A1 = jnp.where(p, s0, s1)
    E1 = jnp.where(p, e0, e1)
    mg = A1 <= E0
    E0 = jnp.where(mg, jnp.maximum(E0, E1), E0)
    A1 = jnp.where(mg, F, A1)
    E1 = jnp.where(mg, F, E1)

    def unit_bgh(n):
        j = n // UPB
        g = n % UPB
        return wid * BPW + j, g * 8

    def grp_full(n):
        g8 = (n % UPB) * 8
        return ((g8 >= A0) & (g8 + 8 <= E0)) | ((g8 >= A1) & (g8 + 8 <= E1))

    def row_masked(f):
        return ((f >= A0) & (f < E0)) | ((f >= A1) & (f < E1))

    def issue_load(n, s):
        b, g8 = unit_bgh(n)
        pltpu.make_async_copy(
            x_hbm.at[b, pl.ds(g8, 8), :], rbufs[s], sin[s]
        ).start()

    def zero_span_vst(row, sa, se, a0, kpos, rem):
        @pl.when(sa < a0)
        def _():
            cs = a0 - 16
            v = row[pl.ds(cs, 16)]
            lane = i16 + cs
            row[pl.ds(cs, 16)] = jnp.where((lane >= sa) & (lane < se),
                                           jnp.float32(0.0), v)

        def _ik4(q, c2):
            base = a0 + q * 64
            for u in range(4):
                row[pl.ds(base + u * 16, 16)] = z16
            return c2
        lax.fori_loop(0, kpos // 4, _ik4, 0)
        kb = a0 + (kpos // 4) * 64
        for u in range(3):
            @pl.when(kpos % 4 > u)
            def _(u=u):
                row[pl.ds(kb + u * 16, 16)] = z16

        t0 = a0 + 16 * kpos

        @pl.when(t0 < se)
        def _():
            cs = jnp.minimum(t0, T - 16)
            v = row[pl.ds(cs, 16)]
            lane = i16 + cs
            row[pl.ds(cs, 16)] = jnp.where((lane >= sa) & (lane < se),
                                           jnp.float32(0.0), v)

    for s in range(NSLOT):
        @pl.when(~grp_full(jnp.int32(s)))
        def _(s=s):
            issue_load(jnp.int32(s), s)

    def iter_body(m, carry):
        o = list(carry[:NSLOT])
        nz = carry[NSLOT]
        for s in range(NSLOT):
            n = m * NSLOT + s

            @pl.when(n < NU)
            def _(n=n, s=s):
                b, g8 = unit_bgh(n)
                j = n // UPB
                xl = sm[SM_XL + j]

                @pl.when(n % UPB == 0)
                def _():
                    def clip(i, c):
                        ts = sm[SM_TS + i]
                        tl = sm[SM_TL + i]
                        sm[SM_SA + i] = jnp.minimum(ts, xl)
                        sm[SM_SE + i] = jnp.minimum(ts + tl, xl)
                        return c
                    lax.fori_loop(0, NTIME, clip, 0)

                full = grp_full(n)

                @pl.when(full)
                def _():
                    pltpu.make_async_copy(
                        zgrp_v, out_hbm.at[b, pl.ds(g8, 8), pl.ds(0, HT)],
                        sem_z).start()
                    pltpu.make_async_copy(
                        zgrp_v, out_hbm.at[b, pl.ds(g8, 8), pl.ds(HT, HT)],
                        sem_z).start()

                @pl.when(~full)
                def _():
                    pltpu.make_async_copy(
                        x_hbm.at[0, pl.ds(0, 8), :], rbufs[s],
                        sin[s]).wait()

                    for r in range(8):
                        @pl.when(row_masked(g8 + r))
                        def _(r=r):
                            row = rbufs[s].at[r]

                            def _zr(kk, c):
                                base = kk * 256
                                for u in range(16):
                                    row[pl.ds(base + u * 16, 16)] = z16
                                return c
                            lax.fori_loop(0, T // 256, _zr, 0)

                    def _sp(i, c):
                        sa = sm[SM_SA + i]
                        se = sm[SM_SE + i]
                        rem = se - sa

                        @pl.when(rem > 0)
                        def _():
                            a0 = ((sa + 15) // 16) * 16
                            kpos = jnp.maximum((se - a0) // 16, 0)
                            for r in range(8):
                                @pl.when(~row_masked(g8 + r))
                                def _(r=r):
                                    zero_span_vst(rbufs[s].at[r], sa, se,
                                                  a0, kpos, rem)
                        return c
                    lax.fori_loop(0, NTIME, _sp, 0)

                    pltpu.make_async_copy(
                        rbufs[s], out_hbm.at[b, pl.ds(g8, 8), :],
                        sout[s]).start()

            nz = nz + jnp.where((n < NU) & grp_full(n), 2, 0)
            o[s] = jnp.where((n < NU) & ~grp_full(n), 1, o[s])

        for s in range(NSLOT):
            n2 = (m + 1) * NSLOT + s

            @pl.when((n2 < NU) & ~grp_full(n2))
            def _(n2=n2, s=s):
                @pl.when(o[s] > 0)
                def _():
                    pltpu.make_async_copy(
                        x_hbm.at[0, pl.ds(0, 8), :], rbufs[s],
                        sout[s]).wait()
                issue_load(n2, s)
            o[s] = jnp.where((n2 < NU) & ~grp_full(n2), 0, o[s])

        return (*o, nz)

    carry = lax.fori_loop(0, NITER, iter_body, (0,) * NSLOT + (0,))

    for s in range(NSLOT):
        @pl.when(carry[s] > 0)
        def _(s=s):
            pltpu.make_async_copy(
                x_hbm.at[0, pl.ds(0, 8), :], rbufs[s], sout[s]
            ).wait()

    def drz(i, c):
        pltpu.make_async_copy(
            x_hbm.at[0, pl.ds(0, 8), pl.ds(0, HT)], zgrp_v, sem_z
        ).wait()
        return c
    lax.fori_loop(0, carry[NSLOT], drz, 0)


def kernel(x, x_len, freq_starts, freq_lengths, time_starts, time_lengths):
    pm = jnp.zeros((35, 16), jnp.int32)
    pm = pm.at[:NW, :BPW].set(x_len.astype(jnp.int32).reshape(NW, BPW))
    pm = pm.at[NW, :NFREQ].set(freq_starts.astype(jnp.int32))
    pm = pm.at[NW, NFREQ:2 * NFREQ].set(freq_lengths.astype(jnp.int32))
    pm = pm.at[NW + 1, :NTIME].set(time_starts.astype(jnp.int32))
    pm = pm.at[NW + 2, :NTIME].set(time_lengths.astype(jnp.int32))
    mesh = plsc.VectorSubcoreMesh(core_axis_name="c", subcore_axis_name="s")
    f = pl.kernel(
        _sc_body,
        out_type=jax.ShapeDtypeStruct((B, F, T), jnp.float32),
        mesh=mesh,
        scratch_types=[
            pltpu.VMEM((35, 16), jnp.int32),
            pltpu.VMEM((8, HT), jnp.float32),
            pltpu.VMEM((8, T), jnp.float32),
            pltpu.VMEM((8, T), jnp.float32),
            pltpu.VMEM((8, T), jnp.float32),
            pltpu.SMEM((64,), jnp.int32),
            pltpu.SemaphoreType.DMA,
            pltpu.SemaphoreType.DMA,
            pltpu.SemaphoreType.DMA,
            pltpu.SemaphoreType.DMA,
            pltpu.SemaphoreType.DMA,
            pltpu.SemaphoreType.DMA,
            pltpu.SemaphoreType.DMA,
        ],
    )
    return f(x, pm)


# speedup vs baseline: 1.6094x; 1.0190x over previous
import jax
import jax.numpy as jnp
from jax import lax
from jax.experimental import pallas as pl
from jax.experimental.pallas import tpu as pltpu
from jax.experimental.pallas import tpu_sc as plsc

B, F, T = 128, 80, 4096
NFREQ, NTIME = 2, 10
NW = 32
BPW = B // NW
GPB = F // 8
HT = T // 2
UPB = GPB
NU = BPW * UPB
NSLOT = 3
NITER = (NU + NSLOT - 1) // NSLOT
SM_XL, SM_TS, SM_TL, SM_SA, SM_SE = 0, 8, 18, 28, 38


def _sc_body(x_hbm, prm_hbm, out_hbm, prm_v, zgrp_v,
             r0, r1, r2, sm,
             si0a, si0b, si1a, si1b, si2a, si2b,
             so0a, so0b, so1a, so1b, so2a, so2b, sem_z):
    cid = lax.axis_index("c")
    sid = lax.axis_index("s")
    wid = sid * 2 + cid

    rbufs = (r0, r1, r2)
    sin = ((si0a, si0b), (si1a, si1b), (si2a, si2b))
    sout = ((so0a, so0b), (so1a, so1b), (so2a, so2b))

    pltpu.sync_copy(prm_hbm, prm_v)

    z16 = jnp.zeros((16,), jnp.float32)
    i16 = lax.broadcasted_iota(jnp.int32, (16,), 0)

    def _zg(i, c):
        zgrp_v[i // (HT // 16), pl.ds((i % (HT // 16)) * 16, 16)] = z16
        return c
    lax.fori_loop(0, 8 * (HT // 16), _zg, 0)

    xlv = prm_v[wid]
    fv = prm_v[32]
    tsv = prm_v[33]
    tlv = prm_v[34]
    for j in range(BPW):
        sm[SM_XL + j] = xlv[j]
    for i in range(NTIME):
        sm[SM_TS + i] = tsv[i]
        sm[SM_TL + i] = tlv[i]

    s0 = fv[0]
    s1 = fv[1]
    e0 = s0 + fv[2]
    e1 = s1 + fv[3]
    p = s1 < s0
    A0 = jnp.where(p, s1, s0)
    E0 = jnp.where(p, e1, e0)
    A1 = jnp.where(p, s0, s1)
    E1 = jnp.where(p, e0, e1)
    mg = A1 <= E0
    E0 = jnp.where(mg, jnp.maximum(E0, E1), E0)
    A1 = jnp.where(mg, F, A1)
    E1 = jnp.where(mg, F, E1)

    def unit_bgh(n):
        j = n // UPB
        g = n % UPB
        return wid * BPW + j, g * 8

    def grp_full(n):
        g8 = (n % UPB) * 8
        return ((g8 >= A0) & (g8 + 8 <= E0)) | ((g8 >= A1) & (g8 + 8 <= E1))

    def row_masked(f):
        return ((f >= A0) & (f < E0)) | ((f >= A1) & (f < E1))

    def issue_load_half(n, s, hh):
        b, g8 = unit_bgh(n)
        pltpu.make_async_copy(
            x_hbm.at[b, pl.ds(g8, 8), pl.ds(hh * HT, HT)],
            rbufs[s].at[:, pl.ds(hh * HT, HT)], sin[s][hh]
        ).start()

    def issue_load(n, s):
        issue_load_half(n, s, 0)
        issue_load_half(n, s, 1)

    def zero_span_vst(row, sa, se, a0, kpos, rem):
        @pl.when(sa < a0)
        def _():
            cs = a0 - 16
            v = row[pl.ds(cs, 16)]
            lane = i16 + cs
            row[pl.ds(cs, 16)] = jnp.where((lane >= sa) & (lane < se),
                                           jnp.float32(0.0), v)

        def _ik4(q, c2):
            base = a0 + q * 64
            for u in range(4):
                row[pl.ds(base + u * 16, 16)] = z16
            return c2
        lax.fori_loop(0, kpos // 4, _ik4, 0)
        kb = a0 + (kpos // 4) * 64
        for u in range(3):
            @pl.when(kpos % 4 > u)
            def _(u=u):
                row[pl.ds(kb + u * 16, 16)] = z16

        t0 = a0 + 16 * kpos

        @pl.when(t0 < se)
        def _():
            cs = jnp.minimum(t0, T - 16)
            v = row[pl.ds(cs, 16)]
            lane = i16 + cs
            row[pl.ds(cs, 16)] = jnp.where((lane >= sa) & (lane < se),
                                           jnp.float32(0.0), v)

    for s in range(NSLOT):
        @pl.when(~grp_full(jnp.int32(s)))
        def _(s=s):
            issue_load(jnp.int32(s), s)

    def iter_body(m, carry):
        o = list(carry[:NSLOT])
        nz = carry[NSLOT]
        for s in range(NSLOT):
            n = m * NSLOT + s

            @pl.when(n < NU)
            def _(n=n, s=s):
                b, g8 = unit_bgh(n)
                j = n // UPB
                xl = sm[SM_XL + j]

                @pl.when(n % UPB == 0)
                def _():
                    def clip(i, c):
                        ts = sm[SM_TS + i]
                        tl = sm[SM_TL + i]
                        sm[SM_SA + i] = jnp.minimum(ts, xl)
                        sm[SM_SE + i] = jnp.minimum(ts + tl, xl)
                        return c
                    lax.fori_loop(0, NTIME, clip, 0)

                full = grp_full(n)

                @pl.when(full)
                def _():
                    pltpu.make_async_copy(
                        zgrp_v, out_hbm.at[b, pl.ds(g8, 8), pl.ds(0, HT)],
                        sem_z).start()
                    pltpu.make_async_copy(
                        zgrp_v, out_hbm.at[b, pl.ds(g8, 8), pl.ds(HT, HT)],
                        sem_z).start()

                @pl.when(~full)
                def _():
                    for hh in range(2):
                        pltpu.make_async_copy(
                            x_hbm.at[0, pl.ds(0, 8), pl.ds(0, HT)],
                            rbufs[s].at[:, pl.ds(0, HT)], sin[s][hh]).wait()

                    for r in range(8):
                        @pl.when(row_masked(g8 + r))
                        def _(r=r):
                            row = rbufs[s].at[r]

                            def _zr(kk, c):
                                base = kk * 256
                                for u in range(16):
                                    row[pl.ds(base + u * 16, 16)] = z16
                                return c
                            lax.fori_loop(0, T // 256, _zr, 0)

                    def _sp(i, c):
                        sa = sm[SM_SA + i]
                        se = sm[SM_SE + i]
                        rem = se - sa

                        @pl.when(rem > 0)
                        def _():
                            a0 = ((sa + 15) // 16) * 16
                            kpos = jnp.maximum((se - a0) // 16, 0)
                            for r in range(8):
                                @pl.when(~row_masked(g8 + r))
                                def _(r=r):
                                    zero_span_vst(rbufs[s].at[r], sa, se,
                                                  a0, kpos, rem)
                        return c
                    lax.fori_loop(0, NTIME, _sp, 0)

                    for hh in range(2):
                        pltpu.make_async_copy(
                            rbufs[s].at[:, pl.ds(hh * HT, HT)],
                            out_hbm.at[b, pl.ds(g8, 8), pl.ds(hh * HT, HT)],
                            sout[s][hh]).start()

            nz = nz + jnp.where((n < NU) & grp_full(n), 2, 0)
            o[s] = jnp.where((n < NU) & ~grp_full(n), 1, o[s])

        for s in range(NSLOT):
            n2 = (m + 1) * NSLOT + s

            @pl.when((n2 < NU) & ~grp_full(n2))
            def _(n2=n2, s=s):
                for hh in range(2):
                    @pl.when(o[s] > 0)
                    def _(hh=hh):
                        pltpu.make_async_copy(
                            x_hbm.at[0, pl.ds(0, 8), pl.ds(0, HT)],
                            rbufs[s].at[:, pl.ds(0, HT)], sout[s][hh]).wait()
                    issue_load_half(n2, s, hh)
            o[s] = jnp.where((n2 < NU) & ~grp_full(n2), 0, o[s])

        return (*o, nz)

    carry = lax.fori_loop(0, NITER, iter_body, (0,) * NSLOT + (0,))

    for s in range(NSLOT):
        @pl.when(carry[s] > 0)
        def _(s=s):
            for hh in range(2):
                pltpu.make_async_copy(
                    x_hbm.at[0, pl.ds(0, 8), pl.ds(0, HT)],
                    rbufs[s].at[:, pl.ds(0, HT)], sout[s][hh]).wait()

    def drz(i, c):
        pltpu.make_async_copy(
            x_hbm.at[0, pl.ds(0, 8), pl.ds(0, HT)], zgrp_v, sem_z
        ).wait()
        return c
    lax.fori_loop(0, carry[NSLOT], drz, 0)


def kernel(x, x_len, freq_starts, freq_lengths, time_starts, time_lengths):
    pm = jnp.zeros((35, 16), jnp.int32)
    pm = pm.at[:NW, :BPW].set(x_len.astype(jnp.int32).reshape(NW, BPW))
    pm = pm.at[NW, :NFREQ].set(freq_starts.astype(jnp.int32))
    pm = pm.at[NW, NFREQ:2 * NFREQ].set(freq_lengths.astype(jnp.int32))
    pm = pm.at[NW + 1, :NTIME].set(time_starts.astype(jnp.int32))
    pm = pm.at[NW + 2, :NTIME].set(time_lengths.astype(jnp.int32))
    mesh = plsc.VectorSubcoreMesh(core_axis_name="c", subcore_axis_name="s")
    f = pl.kernel(
        _sc_body,
        out_type=jax.ShapeDtypeStruct((B, F, T), jnp.float32),
        mesh=mesh,
        scratch_types=[
            pltpu.VMEM((35, 16), jnp.int32),
            pltpu.VMEM((8, HT), jnp.float32),
            pltpu.VMEM((8, T), jnp.float32),
            pltpu.VMEM((8, T), jnp.float32),
            pltpu.VMEM((8, T), jnp.float32),
            pltpu.SMEM((64,), jnp.int32),
            pltpu.SemaphoreType.DMA,
            pltpu.SemaphoreType.DMA,
            pltpu.SemaphoreType.DMA,
            pltpu.SemaphoreType.DMA,
            pltpu.SemaphoreType.DMA,
            pltpu.SemaphoreType.DMA,
            pltpu.SemaphoreType.DMA,
            pltpu.SemaphoreType.DMA,
            pltpu.SemaphoreType.DMA,
            pltpu.SemaphoreType.DMA,
            pltpu.SemaphoreType.DMA,
            pltpu.SemaphoreType.DMA,
            pltpu.SemaphoreType.DMA,
        ],
    )
    return f(x, pm)
